# Initial kernel scaffold; baseline (speedup 1.0000x reference)
import jax
import jax.numpy as jnp
from jax.experimental import pallas as pl


def kernel(x, edge_index, W_fc1, b_fc1, W1, b1, W2, b2, W3, b3, W4, b4, W5, b5, W6, b6, W7, b7):
    raise NotImplementedError("write your pallas kernel here")


# speedup vs baseline: 13.0613x; 13.0613x over previous
import functools

import jax
import jax.numpy as jnp
from jax import lax
from jax.experimental import pallas as pl
from jax.experimental.pallas import tpu as pltpu
from jax.experimental.pallas import tpu_sc as plsc

_N = 10000
_E = 320000
_NC = 2
_NS = 16
_NW = _NC * _NS
_CH = 128
_K = -(-_E // (_NW * _CH))
_EPAD = _NW * _CH * _K
_NPAD = 10240
_ROWS_W = _NPAD // _NS
_DH = 32

@functools.cache
def _sc_mesh():
    return plsc.VectorSubcoreMesh(
        core_axis_name="c", subcore_axis_name="s",
        num_cores=_NC, num_subcores=_NS)


def _deg_body(dst_hbm, ones_hbm, zeros_hbm, out_hbm, dst_v, ones_v, deg_sh):
    c = lax.axis_index("c")
    s = lax.axis_index("s")
    w = c * _NS + s

    @pl.when(s == 0)
    def _():
        pltpu.sync_copy(zeros_hbm, deg_sh)
    pltpu.sync_copy(ones_hbm, ones_v)
    plsc.subcore_barrier()

    def chunk(j, carry):
        pltpu.sync_copy(dst_hbm.at[w * _K + j], dst_v)
        pltpu.sync_copy(ones_v, deg_sh.at[dst_v], add=True)
        return carry

    lax.fori_loop(0, _K, chunk, 0)
    plsc.subcore_barrier()
    pltpu.sync_copy(deg_sh.at[pl.ds(s * _ROWS_W, _ROWS_W)],
                    out_hbm.at[c, pl.ds(s * _ROWS_W, _ROWS_W)])


@functools.cache
def _deg_kernel():
    return pl.kernel(
        _deg_body,
        out_type=jax.ShapeDtypeStruct((_NC, _NPAD, 16), jnp.float32),
        mesh=_sc_mesh(),
        scratch_types=[
            pltpu.VMEM((_CH,), jnp.int32),
            pltpu.VMEM((_CH, 16), jnp.float32),
            pltpu.VMEM_SHARED((_NPAD, 16), jnp.float32),
        ],
        compiler_params=pltpu.CompilerParams(use_tc_tiling_on_sc=False),
    )


def _prop_body(xs_hbm, src_hbm, dst_hbm, zeros_hbm, out_hbm,
               src_v, dst_v, rows_v, s_sh, sem):
    c = lax.axis_index("c")
    s = lax.axis_index("s")
    w = c * _NS + s

    @pl.when(s == 0)
    def _():
        pltpu.sync_copy(zeros_hbm, s_sh)
    plsc.subcore_barrier()

    def chunk(j, carry):
        pltpu.sync_copy(src_hbm.at[w * _K + j], src_v)
        pltpu.sync_copy(dst_hbm.at[w * _K + j], dst_v)
        pltpu.async_copy(xs_hbm.at[src_v], rows_v, sem).wait()
        pltpu.sync_copy(rows_v, s_sh.at[dst_v], add=True)
        return carry

    lax.fori_loop(0, _K, chunk, 0)
    plsc.subcore_barrier()
    pltpu.sync_copy(s_sh.at[pl.ds(s * _ROWS_W, _ROWS_W)],
                    out_hbm.at[c, pl.ds(s * _ROWS_W, _ROWS_W)])


@functools.cache
def _prop_kernel():
    return pl.kernel(
        _prop_body,
        out_type=jax.ShapeDtypeStruct((_NC, _NPAD, _DH), jnp.float32),
        mesh=_sc_mesh(),
        scratch_types=[
            pltpu.VMEM((_CH,), jnp.int32),
            pltpu.VMEM((_CH,), jnp.int32),
            pltpu.VMEM((_CH, _DH), jnp.float32),
            pltpu.VMEM_SHARED((_NPAD, _DH), jnp.float32),
            pltpu.SemaphoreType.DMA,
        ],
        compiler_params=pltpu.CompilerParams(use_tc_tiling_on_sc=False),
    )


def _mynorm(t):
    mn = jnp.min(t, axis=1, keepdims=True)
    mx = jnp.max(t, axis=1, keepdims=True)
    return 2.0 * (t - mn) / (mx - mn + 1e-08) - 1.0


_R = 1024
_G = _NPAD // _R


def _pre_body(x_ref, w_ref, b_ref, degp_ref, x0_ref, xs1_ref, dinv_ref):
    deg = degp_ref[0, :, :1] + degp_ref[1, :, :1] + 1.0
    dinv = lax.rsqrt(deg)
    h = jnp.dot(x_ref[...], w_ref[...], preferred_element_type=jnp.float32)
    h = jnp.maximum(h + b_ref[0], 0.0)
    x0 = _mynorm(h)
    x0_ref[...] = x0
    xs1_ref[...] = x0 * dinv
    dinv_ref[...] = jnp.broadcast_to(dinv, x0.shape)


def _tc_pre(x_pad, w_fc1, b_fc1, degp):
    return pl.pallas_call(
        _pre_body,
        grid=(_G,),
        in_specs=[
            pl.BlockSpec((_R, 128), lambda i: (i, 0)),
            pl.BlockSpec((128, _DH), lambda i: (0, 0)),
            pl.BlockSpec((1, _DH), lambda i: (0, 0)),
            pl.BlockSpec((_NC, _R, 16), lambda i: (0, i, 0)),
        ],
        out_specs=[
            pl.BlockSpec((_R, _DH), lambda i: (i, 0)),
            pl.BlockSpec((_R, _DH), lambda i: (i, 0)),
            pl.BlockSpec((_R, _DH), lambda i: (i, 0)),
        ],
        out_shape=[jax.ShapeDtypeStruct((_NPAD, _DH), jnp.float32)] * 3,
    )(x_pad, w_fc1, b_fc1, degp)


def _post_body(sp_ref, xprev_ref, dinv_ref, w_ref, b_ref, xk_ref, xsn_ref):
    dinv = dinv_ref[...]
    s = sp_ref[0] + sp_ref[1]
    agg = dinv * s + dinv * dinv * xprev_ref[...]
    xk = jnp.dot(agg, w_ref[...], preferred_element_type=jnp.float32) + b_ref[0]
    xk_ref[...] = xk
    xsn_ref[...] = dinv * xk


def _tc_post(sp, xprev, dinv, w, b):
    return pl.pallas_call(
        _post_body,
        grid=(_G,),
        in_specs=[
            pl.BlockSpec((_NC, _R, _DH), lambda i: (0, i, 0)),
            pl.BlockSpec((_R, _DH), lambda i: (i, 0)),
            pl.BlockSpec((_R, _DH), lambda i: (i, 0)),
            pl.BlockSpec((_DH, _DH), lambda i: (0, 0)),
            pl.BlockSpec((1, _DH), lambda i: (0, 0)),
        ],
        out_specs=[
            pl.BlockSpec((_R, _DH), lambda i: (i, 0)),
            pl.BlockSpec((_R, _DH), lambda i: (i, 0)),
        ],
        out_shape=[jax.ShapeDtypeStruct((_NPAD, _DH), jnp.float32)] * 2,
    )(sp, xprev, dinv, w, b)


def _final_body(x0, x1, x2, x3, x4, x5, x6, w_ref, b_ref, out_ref):
    xs = [x0[...], x1[...], x2[...], x3[...], x4[...], x5[...], x6[...]]
    cols = [xs[0], xs[1]]
    for k in range(2, 7):
        cols.append(_mynorm(xs[k]) - _mynorm(xs[k - 2]))
    cat = jnp.concatenate(cols, axis=1)
    out_ref[...] = (
        jnp.dot(cat, w_ref[...], preferred_element_type=jnp.float32) + b_ref[0])


def _tc_final(xlist, w7, b7):
    blk = pl.BlockSpec((_R, _DH), lambda i: (i, 0))
    return pl.pallas_call(
        _final_body,
        grid=(_G,),
        in_specs=[blk] * 7 + [
            pl.BlockSpec((7 * _DH, 128), lambda i: (0, 0)),
            pl.BlockSpec((1, 128), lambda i: (0, 0)),
        ],
        out_specs=pl.BlockSpec((_R, 128), lambda i: (i, 0)),
        out_shape=jax.ShapeDtypeStruct((_NPAD, 128), jnp.float32),
    )(*xlist, w7, b7)


def kernel(x, edge_index, W_fc1, b_fc1, W1, b1, W2, b2, W3, b3, W4, b4,
           W5, b5, W6, b6, W7, b7):
    src = edge_index[0].astype(jnp.int32)
    dst = edge_index[1].astype(jnp.int32)
    pad = _EPAD - _E
    src_p = jnp.concatenate(
        [src, jnp.zeros((pad,), jnp.int32)]).reshape(_NW * _K, _CH)
    dst_p = jnp.concatenate(
        [dst, jnp.full((pad,), _NPAD - 1, jnp.int32)]).reshape(_NW * _K, _CH)

    x_pad = jnp.pad(x, ((0, _NPAD - _N), (0, 0)))
    zeros16 = jnp.zeros((_NPAD, 16), jnp.float32)
    zeros32 = jnp.zeros((_NPAD, _DH), jnp.float32)
    ones16 = jnp.ones((_CH, 16), jnp.float32)

    degp = _deg_kernel()(dst_p, ones16, zeros16)
    x0, xs, dinv = _tc_pre(x_pad, W_fc1, b_fc1.reshape(1, _DH), degp)

    ws = [W1, W2, W3, W4, W5, W6]
    bs = [b1, b2, b3, b4, b5, b6]
    xlist = [x0]
    xprev = x0
    for k in range(6):
        sp = _prop_kernel()(xs, src_p, dst_p, zeros32)
        xk, xs = _tc_post(sp, xprev, dinv, ws[k], bs[k].reshape(1, _DH))
        xlist.append(xk)
        xprev = xk

    out = _tc_final(xlist, W7, b7.reshape(1, 128))
    return out[:_N]
